# baseline (speedup 1.0000x reference)
import functools

import jax
import jax.numpy as jnp
from jax import lax
from jax.experimental import pallas as pl
from jax.experimental.pallas import tpu as pltpu
from jax.experimental.pallas import tpu_sc as plsc

B, S, D = 2, 4096, 1024
E, K, DFF = 64, 2, 1024
N = B * S
CAP = (B * S * K) // E
SLOTS = E * CAP
DUMP = SLOTS
SLOTS_PAD = SLOTS + CAP
TBL_PAD = SLOTS + 16

T = 512
NC = N // T

NSPLIT = 4
EH = E // NSPLIT
HALF = SLOTS // NSPLIT

NCORES, NSUB = 2, 16
NW = NCORES * NSUB
TPW = HALF // NW
TOKW = N // NW
RCH = 32
DRCH = 16


def _routing_body(x_ref, wg_ref, destv_ref, gate_ref, off_ref,
                  ti2_ref, tg2_ref):
    k = pl.program_id(0)
    c = pl.program_id(1)

    @pl.when(jnp.logical_and(k == 0, c == 0))
    def _():
        off_ref[...] = jnp.zeros_like(off_ref)

    @pl.when(k == 0)
    def _():
        x = x_ref[...]
        logits = jnp.dot(x, wg_ref[...], preferred_element_type=jnp.float32)
        m = jnp.max(logits, axis=-1, keepdims=True)
        ex = jnp.exp(logits - m)
        s = ex / jnp.sum(ex, axis=-1, keepdims=True)

        col = lax.broadcasted_iota(jnp.int32, (T, E), 1)
        v1 = jnp.max(s, axis=-1, keepdims=True)
        i1 = jnp.min(jnp.where(s == v1, col, E), axis=-1, keepdims=True)
        m1 = col == i1
        s2 = jnp.where(m1, -1.0, s)
        v2 = jnp.max(s2, axis=-1, keepdims=True)
        i2 = jnp.min(jnp.where(s2 == v2, col, E), axis=-1, keepdims=True)

        denom = (v1 + v2 + 1e-9)[:, 0]
        ti2_ref[pl.ds(c, 1), :] = i2.reshape(1, T)
        tg2_ref[pl.ds(c, 1), :] = (v2[:, 0] / denom).reshape(1, T)
        _routing_tail(destv_ref, gate_ref, off_ref,
                      i1[:, 0], v1[:, 0] / denom)

    @pl.when(k == 1)
    def _():
        i2 = ti2_ref[pl.ds(c, 1), :].reshape(T)
        g2 = tg2_ref[pl.ds(c, 1), :].reshape(T)
        _routing_tail(destv_ref, gate_ref, off_ref, i2, g2)


def _routing_tail(destv_ref, gate_ref, off_ref, idx, gval):
    col = lax.broadcasted_iota(jnp.int32, (T, E), 1)
    onehot = (col == idx[:, None]).astype(jnp.float32)

    r = lax.broadcasted_iota(jnp.int32, (T, T), 0)
    cc = lax.broadcasted_iota(jnp.int32, (T, T), 1)
    tril = (r > cc).astype(jnp.float32)
    excl = jnp.dot(tril, onehot, preferred_element_type=jnp.float32)
    loc = jnp.sum((excl + off_ref[...]) * onehot, axis=-1)

    valid = loc < CAP
    gw = gval * valid.astype(jnp.float32)
    loci = jnp.minimum(loc, CAP - 1).astype(jnp.int32)
    dv = jnp.where(valid, idx * CAP + loci, DUMP)

    destv_ref[...] = dv.reshape(1, 1, T)
    gate_ref[...] = gw.reshape(1, 1, T)
    off_ref[...] += jnp.sum(onehot, axis=0, keepdims=True)


_routing = pl.pallas_call(
    _routing_body,
    grid=(K, NC),
    in_specs=[
        pl.BlockSpec((T, D), lambda k, c: (c, 0)),
        pl.BlockSpec((D, E), lambda k, c: (0, 0)),
    ],
    out_specs=[
        pl.BlockSpec((1, 1, T), lambda k, c: (k * NC + c, 0, 0)),
        pl.BlockSpec((1, 1, T), lambda k, c: (k * NC + c, 0, 0)),
    ],
    out_shape=[
        jax.ShapeDtypeStruct((K * NC, 1, T), jnp.int32),
        jax.ShapeDtypeStruct((K * NC, 1, T), jnp.float32),
    ],
    scratch_shapes=[pltpu.VMEM((1, E), jnp.float32),
                    pltpu.VMEM((NC, T), jnp.int32),
                    pltpu.VMEM((NC, T), jnp.float32)],
    compiler_params=pltpu.CompilerParams(
        dimension_semantics=("arbitrary", "arbitrary")),
)


def _encode_body(half, d0_hbm, d1_hbm, g0_hbm, g1_hbm, x_hbm, disp_hbm,
                 gs_hbm, tbl, gtbl, dall, gall, idxb, rows_a, rows_b,
                 gsem, wsem):
    cid = lax.axis_index("c")
    sid = lax.axis_index("s")
    wid = sid * NCORES + cid
    lbase = wid * TPW
    sbase = half * HALF + lbase

    zi = jnp.zeros((16,), jnp.int32)
    zf = jnp.zeros((16,), jnp.float32)

    def initb(i, _):
        tbl[pl.ds(i * 16, 16)] = zi
        gtbl[pl.ds(i * 16, 16)] = zf
        return 0

    lax.fori_loop(0, (TPW + 16) // 16, initb, 0)

    pltpu.sync_copy(d0_hbm, dall.at[0])
    pltpu.sync_copy(d1_hbm, dall.at[1])
    pltpu.sync_copy(g0_hbm, gall.at[0])
    pltpu.sync_copy(g1_hbm, gall.at[1])
    for li in range(K):

        def body(g, _, li=li):
            idx = dall[li, pl.ds(g * 16, 16)] - sbase
            msk = (idx >= 0) & (idx < TPW)
            idxs = jnp.where(msk, idx, TPW)
            tok = g * 16 + 1 + lax.iota(jnp.int32, 16)
            plsc.store_scatter(tbl, [idxs], tok, mask=msk)
            gv = gall[li, pl.ds(g * 16, 16)]
            plsc.store_scatter(gtbl, [idxs], gv, mask=msk)
            return 0

        lax.fori_loop(0, N // 16, body, 0)

    for chn in range(TPW // RCH):
        for f2 in range(RCH // 16):
            t = tbl[pl.ds(chn * RCH + f2 * 16, 16)]
            idxb[chn, pl.ds(f2 * 16, 16)] = jnp.where(t == 0, 0, t - 1)

    pltpu.sync_copy(gtbl.at[pl.ds(0, TPW)], gs_hbm.at[pl.ds(lbase, TPW)])

    rows = (rows_a, rows_b)
    wd = [None, None]
    for c in range(TPW // RCH):
        b = c % 2
        if c >= 2:
            wd[b].wait()
        pltpu.async_copy(x_hbm.at[idxb.at[c]], rows[b], gsem).wait()
        wd[b] = pltpu.async_copy(
            rows[b], disp_hbm.at[pl.ds(lbase + c * RCH, RCH)], wsem)
    wd[0].wait()
    wd[1].wait()


@functools.lru_cache(maxsize=None)
def _sc_kernels():
    mesh = plsc.VectorSubcoreMesh(
        core_axis_name="c", subcore_axis_name="s",
        num_cores=NCORES, num_subcores=NSUB)

    def make_encode(half, pad):
        return pl.kernel(
            functools.partial(_encode_body, half),
            out_type=[
                jax.ShapeDtypeStruct((HALF + pad, D), jnp.float32),
                jax.ShapeDtypeStruct((HALF,), jnp.float32),
            ],
            mesh=mesh,
            compiler_params=pltpu.CompilerParams(needs_layout_passes=False),
            scratch_types=[
                pltpu.VMEM((TPW + 16,), jnp.int32),
                pltpu.VMEM((TPW + 16,), jnp.float32),
                pltpu.VMEM((K, N), jnp.int32),
                pltpu.VMEM((K, N), jnp.float32),
                pltpu.VMEM((TPW // RCH, RCH), jnp.int32),
                pltpu.VMEM((RCH, D), jnp.float32),
                pltpu.VMEM((RCH, D), jnp.float32),
                pltpu.SemaphoreType.DMA,
                pltpu.SemaphoreType.DMA,
            ],
        )

    encs = tuple(
        make_encode(i, CAP if i == NSPLIT - 1 else 0) for i in range(NSPLIT))
    decode = pl.kernel(
        _decode_body,
        out_type=jax.ShapeDtypeStruct((N, D), jnp.float32),
        mesh=mesh,
        compiler_params=pltpu.CompilerParams(needs_layout_passes=False),
        scratch_types=[
            pltpu.VMEM((TOKW,), jnp.int32),
            pltpu.VMEM((TOKW,), jnp.int32),
            pltpu.VMEM((DRCH, D), jnp.float32),
            pltpu.VMEM((DRCH, D), jnp.float32),
            pltpu.VMEM((DRCH, D), jnp.float32),
            pltpu.VMEM((DRCH, D), jnp.float32),
            pltpu.SemaphoreType.DMA,
            pltpu.SemaphoreType.DMA,
        ],
    )
    return encs, decode


def _ffn_a_body(disp_ref, w1_ref, b1_ref, w2_ref, b2_ref, gs_ref, out_ref):
    xb = disp_ref[...].astype(jnp.bfloat16)
    h = jnp.dot(xb, w1_ref[0].astype(jnp.bfloat16),
                preferred_element_type=jnp.float32)
    h = jnp.maximum(h + b1_ref[0], 0.0)
    y = jnp.dot(h.astype(jnp.bfloat16), w2_ref[0].astype(jnp.bfloat16),
                preferred_element_type=jnp.float32)
    y = y + b2_ref[0]
    out_ref[...] = y * gs_ref[0, 0][:, None]


def _ffn_b_body(yin_ref, disp_ref, w1_ref, b1_ref, w2_ref, b2_ref, gs_ref,
                out_ref):
    del yin_ref
    e = pl.program_id(0)

    @pl.when(e < EH)
    def _():
        xb = disp_ref[...].astype(jnp.bfloat16)
        h = jnp.dot(xb, w1_ref[0].astype(jnp.bfloat16),
                    preferred_element_type=jnp.float32)
        h = jnp.maximum(h + b1_ref[0], 0.0)
        y = jnp.dot(h.astype(jnp.bfloat16), w2_ref[0].astype(jnp.bfloat16),
                    preferred_element_type=jnp.float32)
        y = y + b2_ref[0]
        out_ref[...] = y * gs_ref[0, 0][:, None]

    @pl.when(e >= EH)
    def _():
        out_ref[...] = jnp.zeros_like(out_ref)


_ffn_first = pl.pallas_call(
    _ffn_a_body,
    grid=(EH,),
    in_specs=[
        pl.BlockSpec((CAP, D), lambda e: (e, 0)),
        pl.BlockSpec((1, D, DFF), lambda e: (e, 0, 0)),
        pl.BlockSpec((1, 1, DFF), lambda e: (e, 0, 0)),
        pl.BlockSpec((1, DFF, D), lambda e: (e, 0, 0)),
        pl.BlockSpec((1, 1, D), lambda e: (e, 0, 0)),
        pl.BlockSpec((1, 1, CAP), lambda e: (e, 0, 0)),
    ],
    out_specs=pl.BlockSpec((CAP, D), lambda e: (e, 0)),
    out_shape=jax.ShapeDtypeStruct((SLOTS_PAD, D), jnp.float32),
    compiler_params=pltpu.CompilerParams(
        dimension_semantics=("arbitrary",),
        vmem_limit_bytes=100 * 1024 * 1024),
)


def _make_ffn_part(part):
    base = part * EH
    last = part == NSPLIT - 1

    def wmap(e, base=base):
        return jnp.minimum(e, EH - 1) + base

    return pl.pallas_call(
        _ffn_b_body,
        grid=(EH + 1,) if last else (EH,),
        in_specs=[
            pl.BlockSpec(memory_space=pltpu.MemorySpace.HBM),
            pl.BlockSpec((CAP, D), lambda e: (e, 0)),
            pl.BlockSpec((1, D, DFF), lambda e: (wmap(e), 0, 0)),
            pl.BlockSpec((1, 1, DFF), lambda e: (wmap(e), 0, 0)),
            pl.BlockSpec((1, DFF, D), lambda e: (wmap(e), 0, 0)),
            pl.BlockSpec((1, 1, D), lambda e: (wmap(e), 0, 0)),
            pl.BlockSpec((1, 1, CAP),
                         lambda e: (jnp.minimum(e, EH - 1), 0, 0)),
        ],
        out_specs=pl.BlockSpec((CAP, D), lambda e, base=base: (e + base, 0)),
        out_shape=jax.ShapeDtypeStruct((SLOTS_PAD, D), jnp.float32),
        input_output_aliases={0: 0},
        compiler_params=pltpu.CompilerParams(
            dimension_semantics=("arbitrary",),
            vmem_limit_bytes=100 * 1024 * 1024),
    )


_ffn_parts = tuple(_make_ffn_part(i) for i in range(1, NSPLIT))


def _decode_body(d0_hbm, d1_hbm, yfs_hbm, out_hbm,
                 i0all, i1all, r0a, r0b, r1a, r1b, gsem, wsem):
    cid = lax.axis_index("c")
    sid = lax.axis_index("s")
    wid = sid * NCORES + cid
    tbase = wid * TOKW
    nch = TOKW // DRCH

    pltpu.sync_copy(d0_hbm.at[pl.ds(tbase, TOKW)], i0all)
    pltpu.sync_copy(d1_hbm.at[pl.ds(tbase, TOKW)], i1all)

    r0 = (r0a, r0b)
    r1 = (r1a, r1b)
    wd = [None, None]

    def issue(c, b):
        g0 = pltpu.async_copy(
            yfs_hbm.at[i0all.at[pl.ds(c * DRCH, DRCH)]], r0[b], gsem)
        g1 = pltpu.async_copy(
            yfs_hbm.at[i1all.at[pl.ds(c * DRCH, DRCH)]], r1[b], gsem)
        return g0, g1

    pend = issue(0, 0)
    for c in range(nch):
        b = c % 2
        cur = pend
        if c + 1 < nch:
            if c + 1 >= 2:
                wd[(c + 1) % 2].wait()
            pend = issue(c + 1, (c + 1) % 2)
        cur[0].wait()
        cur[1].wait()

        def row_body(rr, _, b=b):
            def grp(j, _):
                off = j * 16
                r0[b][rr, pl.ds(off, 16)] = (r0[b][rr, pl.ds(off, 16)]
                                             + r1[b][rr, pl.ds(off, 16)])
                return 0

            lax.fori_loop(0, D // 16, grp, 0, unroll=8)
            return 0

        lax.fori_loop(0, DRCH, row_body, 0)
        wd[b] = pltpu.async_copy(
            r0[b], out_hbm.at[pl.ds(tbase + c * DRCH, DRCH)], wsem)
    wd[0].wait()
    wd[1].wait()


def kernel(input, wg, w1, b1, w2, b2):
    xf = input.reshape(N, D)
    destv, gatew = _routing(xf, wg)
    dv = destv.reshape(K, N)
    gw = gatew.reshape(K, N)
    encs, decode = _sc_kernels()
    b1r = b1.reshape(E, 1, DFF)
    b2r = b2.reshape(E, 1, D)
    parts = [enc(dv[0], dv[1], gw[0], gw[1], xf) for enc in encs]
    y = _ffn_first(parts[0][0], w1, b1r, w2, b2r,
                   parts[0][1].reshape(EH, 1, CAP))
    for i in range(1, NSPLIT):
        y = _ffn_parts[i - 1](y, parts[i][0], w1, b1r, w2, b2r,
                              parts[i][1].reshape(EH, 1, CAP))
    out = decode(dv[0], dv[1], y)
    return out.reshape(B, S, D)


# speedup vs baseline: 1.0637x; 1.0637x over previous
import functools

import jax
import jax.numpy as jnp
from jax import lax
from jax.experimental import pallas as pl
from jax.experimental.pallas import tpu as pltpu
from jax.experimental.pallas import tpu_sc as plsc

B, S, D = 2, 4096, 1024
E, K, DFF = 64, 2, 1024
N = B * S
CAP = (B * S * K) // E
SLOTS = E * CAP
DUMP = SLOTS
SLOTS_PAD = SLOTS + CAP
TBL_PAD = SLOTS + 16

T = 512
NC = N // T

NSPLIT = 2
EH = E // NSPLIT
HALF = SLOTS // NSPLIT

NCORES, NSUB = 2, 16
NW = NCORES * NSUB
TPW = HALF // NW
TOKW = N // NW
RCH = 32
DRCH = 16


def _routing_body(x_ref, wg_ref, destv_ref, gate_ref, off_ref,
                  ti2_ref, tg2_ref):
    k = pl.program_id(0)
    c = pl.program_id(1)

    @pl.when(jnp.logical_and(k == 0, c == 0))
    def _():
        off_ref[...] = jnp.zeros_like(off_ref)

    @pl.when(k == 0)
    def _():
        x = x_ref[...]
        logits = jnp.dot(x, wg_ref[...], preferred_element_type=jnp.float32)
        m = jnp.max(logits, axis=-1, keepdims=True)
        ex = jnp.exp(logits - m)
        s = ex / jnp.sum(ex, axis=-1, keepdims=True)

        col = lax.broadcasted_iota(jnp.int32, (T, E), 1)
        v1 = jnp.max(s, axis=-1, keepdims=True)
        i1 = jnp.min(jnp.where(s == v1, col, E), axis=-1, keepdims=True)
        m1 = col == i1
        s2 = jnp.where(m1, -1.0, s)
        v2 = jnp.max(s2, axis=-1, keepdims=True)
        i2 = jnp.min(jnp.where(s2 == v2, col, E), axis=-1, keepdims=True)

        denom = (v1 + v2 + 1e-9)[:, 0]
        ti2_ref[pl.ds(c, 1), :] = i2.reshape(1, T)
        tg2_ref[pl.ds(c, 1), :] = (v2[:, 0] / denom).reshape(1, T)
        _routing_tail(destv_ref, gate_ref, off_ref,
                      i1[:, 0], v1[:, 0] / denom)

    @pl.when(k == 1)
    def _():
        i2 = ti2_ref[pl.ds(c, 1), :].reshape(T)
        g2 = tg2_ref[pl.ds(c, 1), :].reshape(T)
        _routing_tail(destv_ref, gate_ref, off_ref, i2, g2)


def _routing_tail(destv_ref, gate_ref, off_ref, idx, gval):
    col = lax.broadcasted_iota(jnp.int32, (T, E), 1)
    onehot = (col == idx[:, None]).astype(jnp.float32)

    r = lax.broadcasted_iota(jnp.int32, (T, T), 0)
    cc = lax.broadcasted_iota(jnp.int32, (T, T), 1)
    tril = (r > cc).astype(jnp.float32)
    excl = jnp.dot(tril, onehot, preferred_element_type=jnp.float32)
    loc = jnp.sum((excl + off_ref[...]) * onehot, axis=-1)

    valid = loc < CAP
    gw = gval * valid.astype(jnp.float32)
    loci = jnp.minimum(loc, CAP - 1).astype(jnp.int32)
    dv = jnp.where(valid, idx * CAP + loci, DUMP)

    destv_ref[...] = dv.reshape(1, 1, T)
    gate_ref[...] = gw.reshape(1, 1, T)
    off_ref[...] += jnp.sum(onehot, axis=0, keepdims=True)


_routing = pl.pallas_call(
    _routing_body,
    grid=(K, NC),
    in_specs=[
        pl.BlockSpec((T, D), lambda k, c: (c, 0)),
        pl.BlockSpec((D, E), lambda k, c: (0, 0)),
    ],
    out_specs=[
        pl.BlockSpec((1, 1, T), lambda k, c: (k * NC + c, 0, 0)),
        pl.BlockSpec((1, 1, T), lambda k, c: (k * NC + c, 0, 0)),
    ],
    out_shape=[
        jax.ShapeDtypeStruct((K * NC, 1, T), jnp.int32),
        jax.ShapeDtypeStruct((K * NC, 1, T), jnp.float32),
    ],
    scratch_shapes=[pltpu.VMEM((1, E), jnp.float32),
                    pltpu.VMEM((NC, T), jnp.int32),
                    pltpu.VMEM((NC, T), jnp.float32)],
    compiler_params=pltpu.CompilerParams(
        dimension_semantics=("arbitrary", "arbitrary")),
)


def _encode_body(half, d0_hbm, d1_hbm, g0_hbm, g1_hbm, x_hbm, disp_hbm,
                 gs_hbm, tbl, gtbl, dall, gall, idxb, rows_a, rows_b,
                 gsem, wsem):
    cid = lax.axis_index("c")
    sid = lax.axis_index("s")
    wid = sid * NCORES + cid
    lbase = wid * TPW
    sbase = half * HALF + lbase

    zi = jnp.zeros((16,), jnp.int32)
    zf = jnp.zeros((16,), jnp.float32)

    def initb(i, _):
        tbl[pl.ds(i * 16, 16)] = zi
        gtbl[pl.ds(i * 16, 16)] = zf
        return 0

    lax.fori_loop(0, (TPW + 16) // 16, initb, 0)

    pltpu.sync_copy(d0_hbm, dall.at[0])
    pltpu.sync_copy(d1_hbm, dall.at[1])
    pltpu.sync_copy(g0_hbm, gall.at[0])
    pltpu.sync_copy(g1_hbm, gall.at[1])
    for li in range(K):
        @plsc.parallel_loop(0, N // 16, unroll=4)
        def body(g, li=li):
            idx = dall[li, pl.ds(g * 16, 16)] - sbase
            msk = (idx >= 0) & (idx < TPW)
            idxs = jnp.where(msk, idx, TPW)
            tok = g * 16 + 1 + lax.iota(jnp.int32, 16)
            plsc.store_scatter(tbl, [idxs], tok, mask=msk)
            gv = gall[li, pl.ds(g * 16, 16)]
            plsc.store_scatter(gtbl, [idxs], gv, mask=msk)

    for chn in range(TPW // RCH):
        for f2 in range(RCH // 16):
            t = tbl[pl.ds(chn * RCH + f2 * 16, 16)]
            idxb[chn, pl.ds(f2 * 16, 16)] = jnp.where(t == 0, 0, t - 1)

    pltpu.sync_copy(gtbl.at[pl.ds(0, TPW)], gs_hbm.at[pl.ds(lbase, TPW)])

    rows = (rows_a, rows_b)
    wd = [None, None]
    for c in range(TPW // RCH):
        b = c % 2
        if c >= 2:
            wd[b].wait()
        pltpu.async_copy(x_hbm.at[idxb.at[c]], rows[b], gsem).wait()
        wd[b] = pltpu.async_copy(
            rows[b], disp_hbm.at[pl.ds(lbase + c * RCH, RCH)], wsem)
    wd[0].wait()
    wd[1].wait()


@functools.lru_cache(maxsize=None)
def _sc_kernels():
    mesh = plsc.VectorSubcoreMesh(
        core_axis_name="c", subcore_axis_name="s",
        num_cores=NCORES, num_subcores=NSUB)

    def make_encode(half, pad):
        return pl.kernel(
            functools.partial(_encode_body, half),
            out_type=[
                jax.ShapeDtypeStruct((HALF + pad, D), jnp.float32),
                jax.ShapeDtypeStruct((HALF,), jnp.float32),
            ],
            mesh=mesh,
            compiler_params=pltpu.CompilerParams(needs_layout_passes=False),
            scratch_types=[
                pltpu.VMEM((TPW + 16,), jnp.int32),
                pltpu.VMEM((TPW + 16,), jnp.float32),
                pltpu.VMEM((K, N), jnp.int32),
                pltpu.VMEM((K, N), jnp.float32),
                pltpu.VMEM((TPW // RCH, RCH), jnp.int32),
                pltpu.VMEM((RCH, D), jnp.float32),
                pltpu.VMEM((RCH, D), jnp.float32),
                pltpu.SemaphoreType.DMA,
                pltpu.SemaphoreType.DMA,
            ],
        )

    encs = tuple(
        make_encode(i, CAP if i == NSPLIT - 1 else 0) for i in range(NSPLIT))
    decode = pl.kernel(
        _decode_body,
        out_type=jax.ShapeDtypeStruct((N, D), jnp.float32),
        mesh=mesh,
        compiler_params=pltpu.CompilerParams(needs_layout_passes=False),
        scratch_types=[
            pltpu.VMEM((TOKW,), jnp.int32),
            pltpu.VMEM((TOKW,), jnp.int32),
            pltpu.VMEM((DRCH, D), jnp.float32),
            pltpu.VMEM((DRCH, D), jnp.float32),
            pltpu.VMEM((DRCH, D), jnp.float32),
            pltpu.VMEM((DRCH, D), jnp.float32),
            pltpu.SemaphoreType.DMA,
            pltpu.SemaphoreType.DMA,
        ],
    )
    return encs, decode


def _ffn_a_body(disp_ref, w1_ref, b1_ref, w2_ref, b2_ref, gs_ref, out_ref):
    xb = disp_ref[...].astype(jnp.bfloat16)
    h = jnp.dot(xb, w1_ref[0].astype(jnp.bfloat16),
                preferred_element_type=jnp.float32)
    h = jnp.maximum(h + b1_ref[0], 0.0)
    y = jnp.dot(h.astype(jnp.bfloat16), w2_ref[0].astype(jnp.bfloat16),
                preferred_element_type=jnp.float32)
    y = y + b2_ref[0]
    out_ref[...] = y * gs_ref[0, 0][:, None]


def _ffn_b_body(yin_ref, disp_ref, w1_ref, b1_ref, w2_ref, b2_ref, gs_ref,
                out_ref):
    del yin_ref
    e = pl.program_id(0)

    @pl.when(e < EH)
    def _():
        xb = disp_ref[...].astype(jnp.bfloat16)
        h = jnp.dot(xb, w1_ref[0].astype(jnp.bfloat16),
                    preferred_element_type=jnp.float32)
        h = jnp.maximum(h + b1_ref[0], 0.0)
        y = jnp.dot(h.astype(jnp.bfloat16), w2_ref[0].astype(jnp.bfloat16),
                    preferred_element_type=jnp.float32)
        y = y + b2_ref[0]
        out_ref[...] = y * gs_ref[0, 0][:, None]

    @pl.when(e >= EH)
    def _():
        out_ref[...] = jnp.zeros_like(out_ref)


_ffn_first = pl.pallas_call(
    _ffn_a_body,
    grid=(EH,),
    in_specs=[
        pl.BlockSpec((CAP, D), lambda e: (e, 0)),
        pl.BlockSpec((1, D, DFF), lambda e: (e, 0, 0)),
        pl.BlockSpec((1, 1, DFF), lambda e: (e, 0, 0)),
        pl.BlockSpec((1, DFF, D), lambda e: (e, 0, 0)),
        pl.BlockSpec((1, 1, D), lambda e: (e, 0, 0)),
        pl.BlockSpec((1, 1, CAP), lambda e: (e, 0, 0)),
    ],
    out_specs=pl.BlockSpec((CAP, D), lambda e: (e, 0)),
    out_shape=jax.ShapeDtypeStruct((SLOTS_PAD, D), jnp.float32),
    compiler_params=pltpu.CompilerParams(
        dimension_semantics=("arbitrary",),
        vmem_limit_bytes=100 * 1024 * 1024),
)


def _make_ffn_part(part):
    base = part * EH
    last = part == NSPLIT - 1

    def wmap(e, base=base):
        return jnp.minimum(e, EH - 1) + base

    return pl.pallas_call(
        _ffn_b_body,
        grid=(EH + 1,) if last else (EH,),
        in_specs=[
            pl.BlockSpec(memory_space=pltpu.MemorySpace.HBM),
            pl.BlockSpec((CAP, D), lambda e: (e, 0)),
            pl.BlockSpec((1, D, DFF), lambda e: (wmap(e), 0, 0)),
            pl.BlockSpec((1, 1, DFF), lambda e: (wmap(e), 0, 0)),
            pl.BlockSpec((1, DFF, D), lambda e: (wmap(e), 0, 0)),
            pl.BlockSpec((1, 1, D), lambda e: (wmap(e), 0, 0)),
            pl.BlockSpec((1, 1, CAP),
                         lambda e: (jnp.minimum(e, EH - 1), 0, 0)),
        ],
        out_specs=pl.BlockSpec((CAP, D), lambda e, base=base: (e + base, 0)),
        out_shape=jax.ShapeDtypeStruct((SLOTS_PAD, D), jnp.float32),
        input_output_aliases={0: 0},
        compiler_params=pltpu.CompilerParams(
            dimension_semantics=("arbitrary",),
            vmem_limit_bytes=100 * 1024 * 1024),
    )


_ffn_parts = tuple(_make_ffn_part(i) for i in range(1, NSPLIT))


def _decode_body(d0_hbm, d1_hbm, yfs_hbm, out_hbm,
                 i0all, i1all, r0a, r0b, r1a, r1b, gsem, wsem):
    cid = lax.axis_index("c")
    sid = lax.axis_index("s")
    wid = sid * NCORES + cid
    tbase = wid * TOKW
    nch = TOKW // DRCH

    pltpu.sync_copy(d0_hbm.at[pl.ds(tbase, TOKW)], i0all)
    pltpu.sync_copy(d1_hbm.at[pl.ds(tbase, TOKW)], i1all)

    r0 = (r0a, r0b)
    r1 = (r1a, r1b)
    wd = [None, None]

    def issue(c, b):
        g0 = pltpu.async_copy(
            yfs_hbm.at[i0all.at[pl.ds(c * DRCH, DRCH)]], r0[b], gsem)
        g1 = pltpu.async_copy(
            yfs_hbm.at[i1all.at[pl.ds(c * DRCH, DRCH)]], r1[b], gsem)
        return g0, g1

    pend = issue(0, 0)
    for c in range(nch):
        b = c % 2
        cur = pend
        if c + 1 < nch:
            if c + 1 >= 2:
                wd[(c + 1) % 2].wait()
            pend = issue(c + 1, (c + 1) % 2)
        cur[0].wait()
        cur[1].wait()

        def row_body(rr, _, b=b):
            @plsc.parallel_loop(0, D // 16, unroll=8)
            def grp(j):
                off = j * 16
                r0[b][rr, pl.ds(off, 16)] = (r0[b][rr, pl.ds(off, 16)]
                                             + r1[b][rr, pl.ds(off, 16)])

            return 0

        lax.fori_loop(0, DRCH, row_body, 0)
        wd[b] = pltpu.async_copy(
            r0[b], out_hbm.at[pl.ds(tbase + c * DRCH, DRCH)], wsem)
    wd[0].wait()
    wd[1].wait()


def kernel(input, wg, w1, b1, w2, b2):
    xf = input.reshape(N, D)
    destv, gatew = _routing(xf, wg)
    dv = destv.reshape(K, N)
    gw = gatew.reshape(K, N)
    encs, decode = _sc_kernels()
    b1r = b1.reshape(E, 1, DFF)
    b2r = b2.reshape(E, 1, D)
    parts = [enc(dv[0], dv[1], gw[0], gw[1], xf) for enc in encs]
    y = _ffn_first(parts[0][0], w1, b1r, w2, b2r,
                   parts[0][1].reshape(EH, 1, CAP))
    for i in range(1, NSPLIT):
        y = _ffn_parts[i - 1](y, parts[i][0], w1, b1r, w2, b2r,
                              parts[i][1].reshape(EH, 1, CAP))
    out = decode(dv[0], dv[1], y)
    return out.reshape(B, S, D)
